# Initial kernel scaffold; baseline (speedup 1.0000x reference)
import jax
import jax.numpy as jnp
from jax.experimental import pallas as pl


def kernel(pred, gt):
    raise NotImplementedError("write your pallas kernel here")


# speedup vs baseline: 178.1881x; 178.1881x over previous
import jax
import jax.numpy as jnp
from jax import lax
from jax.experimental import pallas as pl
from jax.experimental.pallas import tpu as pltpu

_BLOCK_ROWS = 1024


def _bce(x, z):
    return jnp.maximum(x, 0.0) - x * z + jnp.log1p(jnp.exp(-jnp.abs(x)))


def _main_body(pred_ref, gt_ref, out_ref):
    i = pl.program_id(0)
    x = pred_ref[...]
    z = gt_ref[...]
    loss = _bce(x, z)
    pos_sum = jnp.sum(loss * z)
    pos_cnt = jnp.sum(z)
    neg_sum = jnp.sum(loss * (1.0 - z))

    @pl.when(i == 0)
    def _():
        out_ref[0] = 0.0
        out_ref[1] = 0.0
        out_ref[2] = 0.0

    out_ref[0] += pos_sum
    out_ref[1] += pos_cnt
    out_ref[2] += neg_sum


def _thresh_body(t_ref, pred_ref, gt_ref, out_ref):
    i = pl.program_id(0)
    t = t_ref[0]
    x = pred_ref[...]
    z = gt_ref[...]
    loss = _bce(x, z)
    vals = jnp.where(z == 0.0, loss, -1.0)
    cnt_ge = jnp.sum(jnp.where(vals >= t, 1.0, 0.0))
    cnt_gt = jnp.sum(jnp.where(vals > t, 1.0, 0.0))
    sum_gt = jnp.sum(jnp.where(vals > t, loss, 0.0))

    @pl.when(i == 0)
    def _():
        out_ref[0] = 0.0
        out_ref[1] = 0.0
        out_ref[2] = 0.0

    out_ref[0] += cnt_ge
    out_ref[1] += cnt_gt
    out_ref[2] += sum_gt


def _run_main(p2, g2):
    rows = p2.shape[0]
    grid = rows // _BLOCK_ROWS
    return pl.pallas_call(
        _main_body,
        grid=(grid,),
        in_specs=[
            pl.BlockSpec((_BLOCK_ROWS, p2.shape[1]), lambda i: (i, 0)),
            pl.BlockSpec((_BLOCK_ROWS, p2.shape[1]), lambda i: (i, 0)),
        ],
        out_specs=pl.BlockSpec(memory_space=pltpu.SMEM),
        out_shape=jax.ShapeDtypeStruct((3,), jnp.float32),
    )(p2, g2)


def _run_thresh(p2, g2, t):
    rows = p2.shape[0]
    grid = rows // _BLOCK_ROWS
    return pl.pallas_call(
        _thresh_body,
        grid=(grid,),
        in_specs=[
            pl.BlockSpec(memory_space=pltpu.SMEM),
            pl.BlockSpec((_BLOCK_ROWS, p2.shape[1]), lambda i: (i, 0)),
            pl.BlockSpec((_BLOCK_ROWS, p2.shape[1]), lambda i: (i, 0)),
        ],
        out_specs=pl.BlockSpec(memory_space=pltpu.SMEM),
        out_shape=jax.ShapeDtypeStruct((3,), jnp.float32),
    )(t.reshape(1), p2, g2)


def kernel(pred, gt):
    N, H, W = gt.shape
    total = float(N * H * W)
    eps = 1e-06
    p2 = pred.reshape(N * H, W)
    g2 = gt.reshape(N * H, W)

    sums = _run_main(p2, g2)
    pos_sum, pos_cnt, neg_sum_all = sums[0], sums[1], sums[2]
    neg_cnt_all = total - pos_cnt
    k = jnp.minimum(neg_cnt_all, 3.0 * pos_cnt)

    def fast_fn(_):
        return neg_sum_all

    def rare_fn(_):
        def body(_, carry):
            lo, hi = carry
            mid = lo + (hi - lo) // 2
            t = lax.bitcast_convert_type(mid, jnp.float32)
            c = _run_thresh(p2, g2, t)[0]
            take = c >= k
            return jnp.where(take, mid, lo), jnp.where(take, hi, mid)

        lo0 = jnp.int32(0)
        hi0 = jnp.int32(0x7F800000)
        lo, _hi = lax.fori_loop(0, 31, body, (lo0, hi0))
        t = lax.bitcast_convert_type(lo, jnp.float32)
        o = _run_thresh(p2, g2, t)
        cnt_gt, sum_gt = o[1], o[2]
        return jnp.where(k > 0.0, sum_gt + (k - cnt_gt) * t, 0.0)

    neg_sum = lax.cond(k >= neg_cnt_all, fast_fn, rare_fn, None)
    return (pos_sum + neg_sum) / (pos_cnt + k + eps)
